# baseline (speedup 1.0000x reference)
import functools

import jax
import jax.numpy as jnp
from jax import lax
from jax.experimental import pallas as pl
from jax.experimental.pallas import tpu as pltpu
from jax.experimental.pallas import tpu_sc as plsc

BATCH = 16384
DIM = 16
NUM_WORKERS = 32
B_PER_W = BATCH // NUM_WORKERS
GATHER_CHUNK = 128
N_CHUNKS = B_PER_W // GATHER_CHUNK

TBLK = 131072
PANEL = TBLK // 16
LOG_TBLK = 17
LOG_PANEL = 13


def _pack_panel(x):
    x3 = lax.bitcast_convert_type(x, jnp.int32)
    rnd = jnp.int32(0x8000)
    lo = lax.shift_right_logical(x3[:8] + rnd, 16)
    hi = (x3[8:] + rnd) & jnp.int32(-65536)
    w = lo | hi
    w8 = w.reshape(8, 16, PANEL).swapaxes(0, 1).reshape(128, PANEL)
    return jnp.swapaxes(w8, 0, 1)


def _transpose_body(u_ref, i_ref, uo_ref, io_ref):
    uo_ref[...] = _pack_panel(u_ref[...])
    io_ref[...] = _pack_panel(i_ref[...])


def _relayout(eu_t, ei_t, rows):
    nb = pl.cdiv(rows, TBLK)
    return pl.pallas_call(
        _transpose_body,
        grid=(nb,),
        in_specs=[
            pl.BlockSpec((DIM, TBLK), lambda i: (0, i)),
            pl.BlockSpec((DIM, TBLK), lambda i: (0, i)),
        ],
        out_specs=[
            pl.BlockSpec((PANEL, 128), lambda i: (i, 0)),
            pl.BlockSpec((PANEL, 128), lambda i: (i, 0)),
        ],
        out_shape=[
            jax.ShapeDtypeStruct((nb * PANEL, 128), jnp.int32),
            jax.ShapeDtypeStruct((nb * PANEL, 128), jnp.int32),
        ],
        compiler_params=pltpu.CompilerParams(
            dimension_semantics=("arbitrary",)),
    )(eu_t, ei_t)


HALF = 128
N_ROUNDS = B_PER_W // HALF


def _fire(eu_hbm, ei_hbm, ugidx, igidx, ub, ib, sem, rnd):
    s_abs = pl.ds(rnd * HALF, HALF)
    return [
        pltpu.async_copy(eu_hbm.at[ugidx.at[s_abs]], ub, sem),
        pltpu.async_copy(ei_hbm.at[igidx.at[s_abs]], ib, sem),
    ]


def _gather_body(users_hbm, items_hbm, eu_hbm, ei_hbm, out_hbm,
                 uidx, iidx, ugidx, igidx, ub0, ib0, ub1, ib1,
                 outv, sem0, sem1):
    wid = lax.axis_index("s") * 2 + lax.axis_index("c")
    base = wid * B_PER_W

    pltpu.sync_copy(users_hbm.at[pl.ds(base, B_PER_W)], uidx)
    pltpu.sync_copy(items_hbm.at[pl.ds(base, B_PER_W)], iidx)

    def gidx_step(k, carry):
        sl = pl.ds(pl.multiple_of(k * DIM, DIM), DIM)
        u = uidx[sl]
        i = iidx[sl]
        ugidx[sl] = lax.shift_left(
            lax.shift_right_logical(u, LOG_TBLK), LOG_PANEL) + (u & (PANEL - 1))
        igidx[sl] = lax.shift_left(
            lax.shift_right_logical(i, LOG_TBLK), LOG_PANEL) + (i & (PANEL - 1))
        return carry

    lax.fori_loop(0, B_PER_W // DIM, gidx_step, 0)

    iota16 = lax.iota(jnp.int32, DIM)
    bufs = ((ub0, ib0, sem0), (ub1, ib1, sem1))

    pend = _fire(eu_hbm, ei_hbm, ugidx, igidx, ub0, ib0, sem0, 0)
    for rnd in range(N_ROUNDS):
        ub, ib, _ = bufs[rnd % 2]
        for c in pend:
            c.wait()
        if rnd + 1 < N_ROUNDS:
            nub, nib, nsem = bufs[(rnd + 1) % 2]
            pend = _fire(eu_hbm, ei_hbm, ugidx, igidx, nub, nib, nsem, rnd + 1)

        def chunk(c, carry):
            rel = c * DIM + iota16
            uoff = (lax.shift_right_logical(
                uidx[pl.ds(rnd * HALF + pl.multiple_of(c * DIM, DIM), DIM)],
                LOG_PANEL) & 15) * 8
            ioff = (lax.shift_right_logical(
                iidx[pl.ds(rnd * HALF + pl.multiple_of(c * DIM, DIM), DIM)],
                LOG_PANEL) & 15) * 8
            himask = jnp.full((DIM,), -65536, jnp.int32)
            acc = jnp.zeros((DIM,), jnp.float32)
            for l in range(DIM // 2):
                uw = plsc.load_gather(ub, [rel, uoff + l])
                iw = plsc.load_gather(ib, [rel, ioff + l])
                u_lo = plsc.bitcast(lax.shift_left(uw, 16), jnp.float32)
                u_hi = plsc.bitcast(uw & himask, jnp.float32)
                i_lo = plsc.bitcast(lax.shift_left(iw, 16), jnp.float32)
                i_hi = plsc.bitcast(iw & himask, jnp.float32)
                acc = acc + u_lo * i_lo + u_hi * i_hi
            sig = 1.0 / (1.0 + jnp.exp(-acc))
            outv[pl.ds(rnd * HALF + pl.multiple_of(c * DIM, DIM), DIM)] = sig
            return carry

        lax.fori_loop(0, HALF // DIM, chunk, 0)

    pltpu.sync_copy(outv, out_hbm.at[pl.ds(base, B_PER_W)])


@jax.jit
def _bpr(users, items, embedding_user, embedding_item):
    rows = embedding_user.shape[0]
    eu_lin, ei_lin = _relayout(embedding_user.T, embedding_item.T, rows)

    mesh = plsc.VectorSubcoreMesh(core_axis_name="c", subcore_axis_name="s")
    run = functools.partial(
        pl.kernel,
        out_type=jax.ShapeDtypeStruct((BATCH,), jnp.float32),
        mesh=mesh,
        compiler_params=pltpu.CompilerParams(
            use_tc_tiling_on_sc=False, needs_layout_passes=False),
        scratch_types=[
            pltpu.VMEM((B_PER_W,), jnp.int32),
            pltpu.VMEM((B_PER_W,), jnp.int32),
            pltpu.VMEM((B_PER_W,), jnp.int32),
            pltpu.VMEM((B_PER_W,), jnp.int32),
            pltpu.VMEM((HALF, 128), jnp.int32),
            pltpu.VMEM((HALF, 128), jnp.int32),
            pltpu.VMEM((HALF, 128), jnp.int32),
            pltpu.VMEM((HALF, 128), jnp.int32),
            pltpu.VMEM((B_PER_W,), jnp.float32),
            pltpu.SemaphoreType.DMA,
            pltpu.SemaphoreType.DMA,
        ],
    )(_gather_body)
    return run(users, items, eu_lin, ei_lin)


def kernel(users, items, embedding_user, embedding_item):
    return _bpr(users.astype(jnp.int32), items.astype(jnp.int32),
                embedding_user, embedding_item)


# speedup vs baseline: 1.0042x; 1.0042x over previous
import functools

import jax
import jax.numpy as jnp
from jax import lax
from jax.experimental import pallas as pl
from jax.experimental.pallas import tpu as pltpu
from jax.experimental.pallas import tpu_sc as plsc

BATCH = 16384
DIM = 16
NUM_WORKERS = 32
B_PER_W = BATCH // NUM_WORKERS
GATHER_CHUNK = 128
N_CHUNKS = B_PER_W // GATHER_CHUNK

TBLK = 131072
PANEL = TBLK // 16
LOG_TBLK = 17
LOG_PANEL = 13


def _pack_panel(x):
    x3 = lax.bitcast_convert_type(x, jnp.int32)
    rnd = jnp.int32(0x8000)
    lo = lax.shift_right_logical(x3[:8] + rnd, 16)
    hi = (x3[8:] + rnd) & jnp.int32(-65536)
    w = lo | hi
    w8 = w.reshape(8, 16, PANEL).swapaxes(0, 1).reshape(128, PANEL)
    return jnp.swapaxes(w8, 0, 1)


def _transpose_body(u_ref, i_ref, uo_ref, io_ref):
    uo_ref[...] = _pack_panel(u_ref[...])
    io_ref[...] = _pack_panel(i_ref[...])


def _relayout(eu_t, ei_t, rows):
    nb = pl.cdiv(rows, TBLK)
    return pl.pallas_call(
        _transpose_body,
        grid=(nb,),
        in_specs=[
            pl.BlockSpec((DIM, TBLK), lambda i: (0, i)),
            pl.BlockSpec((DIM, TBLK), lambda i: (0, i)),
        ],
        out_specs=[
            pl.BlockSpec((PANEL, 128), lambda i: (i, 0)),
            pl.BlockSpec((PANEL, 128), lambda i: (i, 0)),
        ],
        out_shape=[
            jax.ShapeDtypeStruct((nb * PANEL, 128), jnp.int32),
            jax.ShapeDtypeStruct((nb * PANEL, 128), jnp.int32),
        ],
        compiler_params=pltpu.CompilerParams(
            dimension_semantics=("arbitrary",)),
    )(eu_t, ei_t)


HALF = 256


def _gather_body(users_hbm, items_hbm, eu_hbm, ei_hbm, out_hbm,
                 uidx, iidx, ugidx, igidx, urows, irows, outv, sem):
    wid = lax.axis_index("s") * 2 + lax.axis_index("c")
    base = wid * B_PER_W

    pltpu.sync_copy(users_hbm.at[pl.ds(base, B_PER_W)], uidx)
    pltpu.sync_copy(items_hbm.at[pl.ds(base, B_PER_W)], iidx)

    def gidx_step(k, carry):
        sl = pl.ds(pl.multiple_of(k * DIM, DIM), DIM)
        u = uidx[sl]
        i = iidx[sl]
        ugidx[sl] = lax.shift_left(
            lax.shift_right_logical(u, LOG_TBLK), LOG_PANEL) + (u & (PANEL - 1))
        igidx[sl] = lax.shift_left(
            lax.shift_right_logical(i, LOG_TBLK), LOG_PANEL) + (i & (PANEL - 1))
        return carry

    lax.fori_loop(0, B_PER_W // DIM, gidx_step, 0)

    iota16 = lax.iota(jnp.int32, DIM)

    for half in range(B_PER_W // HALF):
        copies = []
        for ch in range(HALF // GATHER_CHUNK):
            s_abs = pl.ds(half * HALF + ch * GATHER_CHUNK, GATHER_CHUNK)
            s_buf = pl.ds(ch * GATHER_CHUNK, GATHER_CHUNK)
            copies.append(
                pltpu.async_copy(eu_hbm.at[ugidx.at[s_abs]], urows.at[s_buf], sem))
            copies.append(
                pltpu.async_copy(ei_hbm.at[igidx.at[s_abs]], irows.at[s_buf], sem))
        for c in copies:
            c.wait()

        def chunk(c, carry):
            rel = c * DIM + iota16
            uoff = (lax.shift_right_logical(
                uidx[pl.ds(half * HALF + pl.multiple_of(c * DIM, DIM), DIM)],
                LOG_PANEL) & 15) * 8
            ioff = (lax.shift_right_logical(
                iidx[pl.ds(half * HALF + pl.multiple_of(c * DIM, DIM), DIM)],
                LOG_PANEL) & 15) * 8
            himask = jnp.full((DIM,), -65536, jnp.int32)
            acc = jnp.zeros((DIM,), jnp.float32)
            for l in range(DIM // 2):
                uw = plsc.load_gather(urows, [rel, uoff + l])
                iw = plsc.load_gather(irows, [rel, ioff + l])
                u_lo = plsc.bitcast(lax.shift_left(uw, 16), jnp.float32)
                u_hi = plsc.bitcast(uw & himask, jnp.float32)
                i_lo = plsc.bitcast(lax.shift_left(iw, 16), jnp.float32)
                i_hi = plsc.bitcast(iw & himask, jnp.float32)
                acc = acc + u_lo * i_lo + u_hi * i_hi
            sig = 1.0 / (1.0 + jnp.exp(-acc))
            outv[pl.ds(half * HALF + pl.multiple_of(c * DIM, DIM), DIM)] = sig
            return carry

        lax.fori_loop(0, HALF // DIM, chunk, 0)

    pltpu.sync_copy(outv, out_hbm.at[pl.ds(base, B_PER_W)])


@jax.jit
def _bpr(users, items, embedding_user, embedding_item):
    rows = embedding_user.shape[0]
    eu_lin, ei_lin = _relayout(embedding_user.T, embedding_item.T, rows)

    mesh = plsc.VectorSubcoreMesh(core_axis_name="c", subcore_axis_name="s")
    run = functools.partial(
        pl.kernel,
        out_type=jax.ShapeDtypeStruct((BATCH,), jnp.float32),
        mesh=mesh,
        compiler_params=pltpu.CompilerParams(
            use_tc_tiling_on_sc=False, needs_layout_passes=False),
        scratch_types=[
            pltpu.VMEM((B_PER_W,), jnp.int32),
            pltpu.VMEM((B_PER_W,), jnp.int32),
            pltpu.VMEM((B_PER_W,), jnp.int32),
            pltpu.VMEM((B_PER_W,), jnp.int32),
            pltpu.VMEM((HALF, 128), jnp.int32),
            pltpu.VMEM((HALF, 128), jnp.int32),
            pltpu.VMEM((B_PER_W,), jnp.float32),
            pltpu.SemaphoreType.DMA,
        ],
    )(_gather_body)
    return run(users, items, eu_lin, ei_lin)


def kernel(users, items, embedding_user, embedding_item):
    return _bpr(users.astype(jnp.int32), items.astype(jnp.int32),
                embedding_user, embedding_item)
